# Initial kernel scaffold; baseline (speedup 1.0000x reference)
import jax
import jax.numpy as jnp
from jax.experimental import pallas as pl


def kernel(hidden_states, k_cache, v_cache, seq_lens, Wq, bq, Wkv, bkv, Wo, bo, qn_w, kn_w, ln_w):
    raise NotImplementedError("write your pallas kernel here")


# speedup vs baseline: 4.9190x; 4.9190x over previous
import functools

import jax
import jax.numpy as jnp
from jax import lax
from jax.experimental import pallas as pl
from jax.experimental.pallas import tpu as pltpu
from jax.experimental.pallas import tpu_sc as plsc

B = 32
L = 4096
HKV = 4
G = 4
HQ = 16
D = 128
DM = 2048
DKV = 512
TOPK = 8
CHUNK = 64
WIN = 128
NCH = 64
EPS = 1e-6
SCALE = 1.0 / (D ** 0.5)
HP = lax.Precision.HIGHEST

NS = G * TOPK
NTOK = NS * CHUNK
NROWS_C = B * HKV * NTOK
NROWS_W = B * HKV * WIN
NROWS = NROWS_C + NROWS_W

NW = 32
PW = NROWS // NW
CH_G = 128
NIT = PW // CH_G


def _proj_body(h_ref, wq_ref, wkv_ref, bq_ref, bkv_ref, q_ref, kv_ref):
    h = h_ref[...]
    q_ref[...] = lax.dot_general(h, wq_ref[...], (((1,), (1,)), ((), ())),
                                 precision=HP) + bq_ref[...]
    kv_ref[...] = lax.dot_general(h, wkv_ref[...], (((1,), (1,)), ((), ())),
                                  precision=HP) + bkv_ref[...]


def _proj(h, Wq, bq, Wkv, bkv):
    return pl.pallas_call(
        _proj_body,
        out_shape=[jax.ShapeDtypeStruct((B, DM), jnp.float32),
                   jax.ShapeDtypeStruct((B, 2 * DKV), jnp.float32)],
    )(h, Wq, Wkv, bq.reshape(1, DM), bkv.reshape(1, 2 * DKV))


def _select_body(seq_ref, q_ref, lmk_ref, knew_ref, qnw_ref, lnw_ref,
                 qn_ref, w_ref, idx_ref):
    b = pl.program_id(0)
    sl = seq_ref[b] + 1
    q = q_ref[0]
    qn = qnw_ref[...] * (q * lax.rsqrt(jnp.mean(q * q, -1, keepdims=True) + EPS))
    qn_ref[0] = qn
    lnw = lnw_ref[...]
    knw = knew_ref[0]
    rowio = lax.broadcasted_iota(jnp.int32, (NCH, 1), 0)
    scs = []
    for k in range(HKV):
        lk = lmk_ref[0, k]
        lk = jnp.where(rowio * CHUNK == sl - 1, knw[k:k + 1], lk)
        lkn = lnw * (lk * lax.rsqrt(jnp.mean(lk * lk, -1, keepdims=True) + EPS))
        sc = lax.dot_general(qn[k * G:(k + 1) * G], lkn,
                             (((1,), (1,)), ((), ())), precision=HP)
        scs.append(sc)
    s = jnp.concatenate(scs, axis=0) * SCALE
    colio = lax.broadcasted_iota(jnp.int32, (HQ, NCH), 1)
    s = jnp.where(colio * CHUNK < sl, s, -1e9)
    vals, idxs = [], []
    for _ in range(TOPK):
        m = jnp.max(s, axis=1, keepdims=True)
        am = jnp.min(jnp.where(s == m, colio, NCH), axis=1, keepdims=True)
        vals.append(m)
        idxs.append(am)
        s = jnp.where(colio == am, -jnp.inf, s)
    v8 = jnp.concatenate(vals, axis=1)
    e = jnp.exp(v8)
    w_ref[0] = e / jnp.sum(e, axis=1, keepdims=True)
    idx_ref[0] = jnp.concatenate(idxs, axis=1)


def _select(seq_lens, q4, lmk, k_new, qn_w, ln_w):
    return pl.pallas_call(
        _select_body,
        grid=(B,),
        in_specs=[
            pl.BlockSpec(memory_space=pltpu.SMEM),
            pl.BlockSpec((1, HQ, D), lambda b: (b, 0, 0)),
            pl.BlockSpec((1, HKV, NCH, D), lambda b: (b, 0, 0, 0)),
            pl.BlockSpec((1, HKV, D), lambda b: (b, 0, 0)),
            pl.BlockSpec((1, D), lambda b: (0, 0)),
            pl.BlockSpec((1, D), lambda b: (0, 0)),
        ],
        out_specs=[
            pl.BlockSpec((1, HQ, D), lambda b: (b, 0, 0)),
            pl.BlockSpec((1, HQ, TOPK), lambda b: (b, 0, 0)),
            pl.BlockSpec((1, HQ, TOPK), lambda b: (b, 0, 0)),
        ],
        out_shape=[jax.ShapeDtypeStruct((B, HQ, D), jnp.float32),
                   jax.ShapeDtypeStruct((B, HQ, TOPK), jnp.float32),
                   jax.ShapeDtypeStruct((B, HQ, TOPK), jnp.int32)],
    )(seq_lens, q4, lmk, k_new, qn_w, ln_w)


def _sc_gather(k2, v2, gidx):
    mesh = plsc.VectorSubcoreMesh(core_axis_name="c", subcore_axis_name="s")

    @functools.partial(
        pl.kernel,
        out_type=[jax.ShapeDtypeStruct((NROWS, D), jnp.float32),
                  jax.ShapeDtypeStruct((NROWS, D), jnp.float32)],
        mesh=mesh,
        scratch_types=[pltpu.VMEM((PW,), jnp.int32),
                       pltpu.VMEM((CH_G, D), jnp.float32),
                       pltpu.VMEM((CH_G, D), jnp.float32),
                       pltpu.SemaphoreType.DMA,
                       pltpu.SemaphoreType.DMA],
    )
    def kern(k_hbm, v_hbm, i_hbm, ok_hbm, ov_hbm, idx_v, kb, vb, gsemk, gsemv):
        wid = lax.axis_index("s") * 2 + lax.axis_index("c")
        base = wid * PW
        pltpu.sync_copy(i_hbm.at[pl.ds(base, PW)], idx_v)

        @pl.loop(0, NIT)
        def _(it):
            isl = idx_v.at[pl.ds(it * CH_G, CH_G)]
            hk = pltpu.async_copy(k_hbm.at[isl], kb, gsemk)
            hv = pltpu.async_copy(v_hbm.at[isl], vb, gsemv)
            hk.wait()
            hv.wait()
            pltpu.sync_copy(kb, ok_hbm.at[pl.ds(base + it * CH_G, CH_G)])
            pltpu.sync_copy(vb, ov_hbm.at[pl.ds(base + it * CH_G, CH_G)])

    return kern(k2, v2, gidx)


def _attn_body(seq_ref, qn_ref, knew_ref, vnew_ref, knw_ref, idxf_ref, wch_ref,
               kgc_ref, vgc_ref, kgw_ref, vgw_ref, o_ref):
    b = pl.program_id(0)
    sl = seq_ref[b] + 1
    slf = sl.astype(jnp.float32)
    qn = qn_ref[0, 0]
    qk = qn * knw_ref[...]
    knew = knew_ref[0, 0]
    vnew = vnew_ref[0, 0]

    kc = kgc_ref[...]
    vc = vgc_ref[...]
    idxf = idxf_ref[...]
    rio = lax.broadcasted_iota(jnp.int32, (NS, NTOK), 0)
    cio = lax.broadcasted_iota(jnp.int32, (NS, NTOK), 1)
    S = (rio == cio // CHUNK).astype(jnp.float32)
    chpos = lax.dot_general(S, idxf, (((0,), (0,)), ((), ())),
                            precision=HP)
    sub = lax.broadcasted_iota(jnp.int32, (NTOK, 1), 0)
    cmod = sub - (sub // CHUNK) * CHUNK
    pos = chpos * float(CHUNK) + cmod.astype(jnp.float32)
    valid = (pos < slf).astype(jnp.float32)
    isnew = pos == (slf - 1.0)
    kc = jnp.where(isnew, knew, kc)
    vc = jnp.where(isnew, vnew, vc)
    ss = jnp.sum(kc * kc, axis=1, keepdims=True)
    rinv = lax.rsqrt(ss * (1.0 / D) + EPS)
    l4 = lax.dot_general(kc, qk, (((1,), (1,)), ((), ())), precision=HP)
    gsel = (lax.broadcasted_iota(jnp.int32, (NTOK, G), 1)
            == lax.broadcasted_iota(jnp.int32, (NTOK, G), 0) // (TOPK * CHUNK)
            ).astype(jnp.float32)
    logit = jnp.sum(l4 * gsel, axis=1, keepdims=True) * rinv * SCALE
    e = jnp.exp(logit) * valid
    d32 = lax.dot_general(S, e, (((1,), (0,)), ((), ())), precision=HP)
    dcol = lax.dot_general(S, d32, (((0,), (0,)), ((), ())), precision=HP)
    attn = e / (dcol + 1e-30)
    o32 = lax.dot_general(S, attn * vc, (((1,), (0,)), ((), ())), precision=HP)
    wch = wch_ref[0, 0]
    w32 = jnp.concatenate([wch] * G, axis=1)
    wm = (lax.broadcasted_iota(jnp.int32, (G, NS), 1) // TOPK
          == lax.broadcasted_iota(jnp.int32, (G, NS), 0)).astype(jnp.float32)
    o_hsa = lax.dot_general(w32 * wm, o32, (((1,), (0,)), ((), ())),
                            precision=HP)

    kw_ = kgw_ref[...]
    vw_ = vgw_ref[...]
    wsub = lax.broadcasted_iota(jnp.int32, (WIN, 1), 0)
    wstart = jnp.maximum(sl - WIN, 0)
    wpos = wsub + wstart
    wvalid = (wpos < sl).astype(jnp.float32)
    wnew = wpos == sl - 1
    kw_ = jnp.where(wnew, knew, kw_)
    vw_ = jnp.where(wnew, vnew, vw_)
    ssw = jnp.sum(kw_ * kw_, axis=1, keepdims=True)
    rinvw = lax.rsqrt(ssw * (1.0 / D) + EPS)
    lw = lax.dot_general(kw_, qk, (((1,), (1,)), ((), ())),
                         precision=HP) * (rinvw * SCALE)
    ew = jnp.exp(lw) * wvalid
    dw = jnp.sum(ew, axis=0, keepdims=True)
    aw = ew / dw
    o_swa = lax.dot_general(aw, vw_, (((0,), (0,)), ((), ())), precision=HP)
    o_ref[0, 0] = o_hsa + o_swa


def _attn(seq_lens, qn4, knew4, vnew4, kn_w, idxf, wch4, kg, vg):
    return pl.pallas_call(
        _attn_body,
        grid=(B, HKV),
        in_specs=[
            pl.BlockSpec(memory_space=pltpu.SMEM),
            pl.BlockSpec((1, 1, G, D), lambda b, k: (b, k, 0, 0)),
            pl.BlockSpec((1, 1, 1, D), lambda b, k: (b, k, 0, 0)),
            pl.BlockSpec((1, 1, 1, D), lambda b, k: (b, k, 0, 0)),
            pl.BlockSpec((1, D), lambda b, k: (0, 0)),
            pl.BlockSpec((NS, 1), lambda b, k: (b * HKV + k, 0)),
            pl.BlockSpec((1, 1, G, TOPK), lambda b, k: (b, k, 0, 0)),
            pl.BlockSpec((NTOK, D), lambda b, k: (b * HKV + k, 0)),
            pl.BlockSpec((NTOK, D), lambda b, k: (b * HKV + k, 0)),
            pl.BlockSpec((WIN, D), lambda b, k: (NROWS_C // WIN + b * HKV + k, 0)),
            pl.BlockSpec((WIN, D), lambda b, k: (NROWS_C // WIN + b * HKV + k, 0)),
        ],
        out_specs=pl.BlockSpec((1, 1, G, D), lambda b, k: (b, k, 0, 0)),
        out_shape=jax.ShapeDtypeStruct((B, HKV, G, D), jnp.float32),
    )(seq_lens, qn4, knew4, vnew4, kn_w, idxf, wch4, kg, vg, kg, vg)


def _out_body(o_ref, wo_ref, bo_ref, out_ref):
    out_ref[...] = lax.dot_general(o_ref[...], wo_ref[...],
                                   (((1,), (1,)), ((), ())),
                                   precision=HP) + bo_ref[...]


def _outproj(o2, Wo, bo):
    return pl.pallas_call(
        _out_body,
        out_shape=jax.ShapeDtypeStruct((B, DM), jnp.float32),
    )(o2, Wo, bo.reshape(1, DM))


def kernel(hidden_states, k_cache, v_cache, seq_lens, Wq, bq, Wkv, bkv,
           Wo, bo, qn_w, kn_w, ln_w):
    h = hidden_states[:, 0, :]
    q_r, kv = _proj(h, Wq, bq, Wkv, bkv)
    q4 = q_r.reshape(B, HQ, D)
    k_new = kv[:, :DKV].reshape(B, HKV, D)
    v_new = kv[:, DKV:].reshape(B, HKV, D)
    lmk = jnp.transpose(k_cache[:, ::CHUNK], (0, 2, 1, 3))
    qn, wch, idxc = _select(seq_lens, q4, lmk, k_new,
                            qn_w.reshape(1, D), ln_w.reshape(1, D))

    bb = jnp.arange(B, dtype=jnp.int32)
    pos = idxc[..., None] * CHUNK + jnp.arange(CHUNK, dtype=jnp.int32)
    kofhq = (jnp.arange(HQ, dtype=jnp.int32) // G)[None, :, None, None]
    rows_c = (bb[:, None, None, None] * L + pos) * HKV + kofhq
    sl = seq_lens + 1
    wstart = jnp.maximum(sl - WIN, 0)
    wpos = wstart[:, None] + jnp.arange(WIN, dtype=jnp.int32)[None, :]
    rows_w = ((bb[:, None, None] * L + wpos[:, None, :]) * HKV
              + jnp.arange(HKV, dtype=jnp.int32)[None, :, None])
    gidx = jnp.concatenate([rows_c.reshape(-1), rows_w.reshape(-1)])

    kg, vg = _sc_gather(k_cache.reshape(-1, D), v_cache.reshape(-1, D), gidx)

    o = _attn(seq_lens, qn.reshape(B, HKV, G, D),
              k_new.reshape(B, HKV, 1, D), v_new.reshape(B, HKV, 1, D),
              kn_w.reshape(1, D),
              idxc.astype(jnp.float32).reshape(B * HQ * TOPK, 1),
              wch.reshape(B, HKV, G, TOPK), kg, vg)
    out = _outproj(o.reshape(B, DM), Wo, bo)
    return out[:, None, :]


# speedup vs baseline: 7.2525x; 1.4744x over previous
import functools

import jax
import jax.numpy as jnp
from jax import lax
from jax.experimental import pallas as pl
from jax.experimental.pallas import tpu as pltpu
from jax.experimental.pallas import tpu_sc as plsc

B = 32
L = 4096
HKV = 4
G = 4
HQ = 16
D = 128
DM = 2048
DKV = 512
TOPK = 8
CHUNK = 64
WIN = 128
NCH = 64
EPS = 1e-6
SCALE = 1.0 / (D ** 0.5)
HP = lax.Precision.HIGHEST

NS = G * TOPK
NTOK = NS * CHUNK
NROWS_C = B * HKV * NTOK
NROWS_W = B * HKV * WIN
NROWS = NROWS_C + NROWS_W

NW = 32
PW = NROWS // NW
CH_G = 128
NIT = PW // CH_G


def _prsqrt(x):
    y = lax.rsqrt(x)
    return y * (1.5 - 0.5 * x * y * y)


def _proj_body(h_ref, wq_ref, wkv_ref, bq_ref, bkv_ref, q_ref, kv_ref):
    h = h_ref[...]
    q_ref[...] = lax.dot_general(h, wq_ref[...], (((1,), (1,)), ((), ())),
                                 precision=HP) + bq_ref[...]
    kv_ref[...] = lax.dot_general(h, wkv_ref[...], (((1,), (1,)), ((), ())),
                                  precision=HP) + bkv_ref[...]


def _proj(h, Wq, bq, Wkv, bkv):
    return pl.pallas_call(
        _proj_body,
        out_shape=[jax.ShapeDtypeStruct((B, DM), jnp.float32),
                   jax.ShapeDtypeStruct((B, 2 * DKV), jnp.float32)],
    )(h, Wq, Wkv, bq.reshape(1, DM), bkv.reshape(1, 2 * DKV))


def _select_body(seq_ref, q_ref, lmk_ref, knew_ref, qnw_ref, lnw_ref,
                 qn_ref, w_ref, idx_ref):
    b = pl.program_id(0)
    sl = seq_ref[b] + 1
    q = q_ref[0]
    qn = qnw_ref[...] * (q * _prsqrt(jnp.mean(q * q, -1, keepdims=True) + EPS))
    qn_ref[0] = qn
    lnw = lnw_ref[...]
    knw = knew_ref[0]
    rowio = lax.broadcasted_iota(jnp.int32, (NCH, 1), 0)
    cols = []
    for k in range(HKV):
        lk = lmk_ref[0, k]
        lk = jnp.where(rowio * CHUNK == sl - 1, knw[k:k + 1], lk)
        lkn = lnw * (lk * _prsqrt(jnp.mean(lk * lk, -1, keepdims=True) + EPS))
        for g in range(G):
            cols.append(jnp.sum(lkn * qn[k * G + g:k * G + g + 1],
                                axis=1, keepdims=True))
    s = jnp.concatenate(cols, axis=1) * SCALE
    s = jnp.where(rowio * CHUNK < sl, s, -1e9)
    vals, idxs = [], []
    for _ in range(TOPK):
        m = jnp.max(s, axis=0, keepdims=True)
        am = jnp.min(jnp.where(s == m, rowio, NCH), axis=0, keepdims=True)
        vals.append(m)
        idxs.append(am)
        s = jnp.where(rowio == am, -jnp.inf, s)
    v8 = jnp.concatenate(vals, axis=0)
    e = jnp.exp(v8)
    w_ref[0] = e / jnp.sum(e, axis=0, keepdims=True)
    idx_ref[0] = jnp.concatenate(idxs, axis=0)


def _select(seq_lens, q4, lmk, k_new, qn_w, ln_w):
    return pl.pallas_call(
        _select_body,
        grid=(B,),
        in_specs=[
            pl.BlockSpec(memory_space=pltpu.SMEM),
            pl.BlockSpec((1, HQ, D), lambda b: (b, 0, 0)),
            pl.BlockSpec((1, HKV, NCH, D), lambda b: (b, 0, 0, 0)),
            pl.BlockSpec((1, HKV, D), lambda b: (b, 0, 0)),
            pl.BlockSpec((1, D), lambda b: (0, 0)),
            pl.BlockSpec((1, D), lambda b: (0, 0)),
        ],
        out_specs=[
            pl.BlockSpec((1, HQ, D), lambda b: (b, 0, 0)),
            pl.BlockSpec((1, TOPK, HQ), lambda b: (b, 0, 0)),
            pl.BlockSpec((1, TOPK, HQ), lambda b: (b, 0, 0)),
        ],
        out_shape=[jax.ShapeDtypeStruct((B, HQ, D), jnp.float32),
                   jax.ShapeDtypeStruct((B, TOPK, HQ), jnp.float32),
                   jax.ShapeDtypeStruct((B, TOPK, HQ), jnp.int32)],
    )(seq_lens, q4, lmk, k_new, qn_w, ln_w)


def _sc_gather(k2, v2, gidx):
    mesh = plsc.VectorSubcoreMesh(core_axis_name="c", subcore_axis_name="s")

    @functools.partial(
        pl.kernel,
        out_type=[jax.ShapeDtypeStruct((NROWS, D), jnp.float32),
                  jax.ShapeDtypeStruct((NROWS, D), jnp.float32)],
        mesh=mesh,
        scratch_types=[pltpu.VMEM((PW,), jnp.int32),
                       pltpu.VMEM((CH_G, D), jnp.float32),
                       pltpu.VMEM((CH_G, D), jnp.float32),
                       pltpu.SemaphoreType.DMA,
                       pltpu.SemaphoreType.DMA],
    )
    def kern(k_hbm, v_hbm, i_hbm, ok_hbm, ov_hbm, idx_v, kb, vb, gsemk, gsemv):
        wid = lax.axis_index("s") * 2 + lax.axis_index("c")
        base = wid * PW
        pltpu.sync_copy(i_hbm.at[pl.ds(base, PW)], idx_v)

        @pl.loop(0, NIT)
        def _(it):
            isl = idx_v.at[pl.ds(it * CH_G, CH_G)]
            hk = pltpu.async_copy(k_hbm.at[isl], kb, gsemk)
            hv = pltpu.async_copy(v_hbm.at[isl], vb, gsemv)
            hk.wait()
            hv.wait()
            pltpu.sync_copy(kb, ok_hbm.at[pl.ds(base + it * CH_G, CH_G)])
            pltpu.sync_copy(vb, ov_hbm.at[pl.ds(base + it * CH_G, CH_G)])

    return kern(k2, v2, gidx)


def _attn_body(seq_ref, qn_ref, knew_ref, vnew_ref, knw_ref, qnw_ref,
               idxf_ref, wch_ref, kgc_ref, vgc_ref, kgw_ref, vgw_ref, o_ref):
    b = pl.program_id(0)
    sl = seq_ref[b] + 1
    slf = sl.astype(jnp.float32)
    qr = qn_ref[0, 0]
    qn = qnw_ref[...] * (qr * _prsqrt(jnp.mean(qr * qr, -1, keepdims=True)
                                      + EPS))
    qk = qn * knw_ref[...]
    knew = knew_ref[0, 0]
    vnew = vnew_ref[0, 0]
    lnew = lax.dot_general(qk, knew, (((1,), (1,)), ((), ())),
                           precision=HP)
    ssnew = jnp.sum(knew * knew, axis=1, keepdims=True)

    kc = kgc_ref[...]
    vc = vgc_ref[...]
    idxf = idxf_ref[0]
    rio = lax.broadcasted_iota(jnp.int32, (NS, NTOK), 0)
    cio = lax.broadcasted_iota(jnp.int32, (NS, NTOK), 1)
    S = (rio == cio // CHUNK).astype(jnp.float32)
    lane = lax.broadcasted_iota(jnp.int32, (1, NTOK), 1)
    cmod = (lane - (lane // CHUNK) * CHUNK).astype(jnp.float32)
    pos = lax.dot_general(idxf, S, (((1,), (0,)), ((), ())),
                          precision=HP) * float(CHUNK) + cmod
    valid = (pos < slf).astype(jnp.float32)
    isnew = (pos == slf - 1.0).astype(jnp.float32)
    gm = (lax.broadcasted_iota(jnp.int32, (G, NTOK), 1) // (TOPK * CHUNK)
          == lax.broadcasted_iota(jnp.int32, (G, NTOK), 0)
          ).astype(jnp.float32)
    l4 = lax.dot_general(qk, kc, (((1,), (1,)), ((), ())),
                         precision=HP)
    lc = jnp.sum(l4 * gm, axis=0, keepdims=True)
    lnewc = jnp.sum(lnew * gm, axis=0, keepdims=True)
    lc = lc + isnew * (lnewc - lc)
    ones = jnp.ones((1, D), dtype=jnp.float32)
    ss = lax.dot_general(ones, kc * kc, (((1,), (1,)), ((), ())),
                         precision=HP)
    ss = ss + isnew * (ssnew - ss)
    rinv = _prsqrt(ss * (1.0 / D) + EPS)
    ec = jnp.exp(lc * rinv * SCALE) * valid
    d32 = lax.dot_general(ec, S, (((1,), (1,)), ((), ())), precision=HP)
    dlane = lax.dot_general(d32, S, (((1,), (0,)), ((), ())), precision=HP)
    attn = ec / (dlane + 1e-30)
    tcol = jnp.transpose(isnew, (1, 0))
    vcp = vc + tcol * (vnew - vc)
    o32 = lax.dot_general(S * attn, vcp, (((1,), (0,)), ((), ())),
                          precision=HP)
    wch = wch_ref[0, 0]
    w32 = jnp.concatenate([wch] * G, axis=1)
    wm = (lax.broadcasted_iota(jnp.int32, (G, NS), 1) // TOPK
          == lax.broadcasted_iota(jnp.int32, (G, NS), 0)).astype(jnp.float32)
    o_hsa = lax.dot_general(w32 * wm, o32, (((1,), (0,)), ((), ())),
                            precision=HP)

    kw_ = kgw_ref[...]
    vw_ = vgw_ref[...]
    wlane = lax.broadcasted_iota(jnp.int32, (1, WIN), 1)
    wstart = jnp.maximum(sl - WIN, 0)
    wpos = wlane + wstart
    wvalid = (wpos < sl).astype(jnp.float32)
    wisnew = (wpos == sl - 1).astype(jnp.float32)
    lw = lax.dot_general(qk, kw_, (((1,), (1,)), ((), ())),
                         precision=HP)
    lw = lw + wisnew * (lnew - lw)
    ssw = lax.dot_general(ones, kw_ * kw_, (((1,), (1,)), ((), ())),
                          precision=HP)
    ssw = ssw + wisnew * (ssnew - ssw)
    rinvw = _prsqrt(ssw * (1.0 / D) + EPS)
    ew = jnp.exp(lw * rinvw * SCALE) * wvalid
    dw = jnp.sum(ew, axis=1, keepdims=True)
    aw = ew / dw
    twin = jnp.transpose(wisnew, (1, 0))
    vwp = vw_ + twin * (vnew - vw_)
    o_swa = lax.dot_general(aw, vwp, (((1,), (0,)), ((), ())), precision=HP)
    o_ref[0, 0] = o_hsa + o_swa


def _attn(seq_lens, qn4, knew4, vnew4, kn_w, qn_w, idxf, wch4, kg, vg):
    return pl.pallas_call(
        _attn_body,
        grid=(B, HKV),
        in_specs=[
            pl.BlockSpec(memory_space=pltpu.SMEM),
            pl.BlockSpec((1, 1, G, D), lambda b, k: (b, k, 0, 0)),
            pl.BlockSpec((1, 1, 1, D), lambda b, k: (b, k, 0, 0)),
            pl.BlockSpec((1, 1, 1, D), lambda b, k: (b, k, 0, 0)),
            pl.BlockSpec((1, D), lambda b, k: (0, 0)),
            pl.BlockSpec((1, D), lambda b, k: (0, 0)),
            pl.BlockSpec((1, 1, NS), lambda b, k: (b * HKV + k, 0, 0)),
            pl.BlockSpec((1, 1, G, TOPK), lambda b, k: (b, k, 0, 0)),
            pl.BlockSpec((NTOK, D), lambda b, k: (b * HKV + k, 0)),
            pl.BlockSpec((NTOK, D), lambda b, k: (b * HKV + k, 0)),
            pl.BlockSpec((WIN, D), lambda b, k: (NROWS_C // WIN + b * HKV + k, 0)),
            pl.BlockSpec((WIN, D), lambda b, k: (NROWS_C // WIN + b * HKV + k, 0)),
        ],
        out_specs=pl.BlockSpec((1, 1, G, D), lambda b, k: (b, k, 0, 0)),
        out_shape=jax.ShapeDtypeStruct((B, HKV, G, D), jnp.float32),
    )(seq_lens, qn4, knew4, vnew4, kn_w, qn_w, idxf, wch4, kg, vg, kg, vg)


def _out_body(o_ref, wo_ref, bo_ref, out_ref):
    out_ref[...] = lax.dot_general(o_ref[...], wo_ref[...],
                                   (((1,), (1,)), ((), ())),
                                   precision=HP) + bo_ref[...]


def _outproj(o2, Wo, bo):
    return pl.pallas_call(
        _out_body,
        out_shape=jax.ShapeDtypeStruct((B, DM), jnp.float32),
    )(o2, Wo, bo.reshape(1, DM))


def kernel(hidden_states, k_cache, v_cache, seq_lens, Wq, bq, Wkv, bkv,
           Wo, bo, qn_w, kn_w, ln_w):
    h = hidden_states[:, 0, :]
    q_r, kv = _proj(h, Wq, bq, Wkv, bkv)
    q4 = q_r.reshape(B, HQ, D)
    k_new = kv[:, :DKV].reshape(B, HKV, D)
    v_new = kv[:, DKV:].reshape(B, HKV, D)

    def _rms(x, w):
        xf = x.astype(jnp.float32)
        return w * (xf * lax.rsqrt(jnp.mean(xf * xf, -1, keepdims=True) + EPS))

    qs = _rms((h @ Wq.T + bq).reshape(B, HKV, G, D), qn_w)
    kvs = h @ Wkv.T + bkv
    kns = kvs[:, :DKV].reshape(B, HKV, D)
    js = jnp.arange(NCH)
    lmks = k_cache[:, ::CHUNK]
    lmks = jnp.where(((js * CHUNK)[None, :] == seq_lens[:, None])[..., None, None],
                     kns[:, None], lmks)
    lmks = _rms(lmks, ln_w)
    scores = jnp.einsum('bkgd,bckd->bkgc', qs, lmks) * SCALE
    scores = jnp.where(((js * CHUNK)[None, :] < (seq_lens + 1)[:, None])
                       [:, None, None, :], scores, -1e9)
    vals, idx_s = jax.lax.top_k(scores, TOPK)
    wsel = jax.nn.softmax(vals, axis=-1)
    wch = wsel.reshape(B, HQ, TOPK)
    idxc = idx_s.astype(jnp.int32).reshape(B, HQ, TOPK)

    bb = jnp.arange(B, dtype=jnp.int32)
    pos = idxc[..., None] * CHUNK + jnp.arange(CHUNK, dtype=jnp.int32)
    kofhq = (jnp.arange(HQ, dtype=jnp.int32) // G)[None, :, None, None]
    rows_c = (bb[:, None, None, None] * L + pos) * HKV + kofhq
    sl = seq_lens + 1
    wstart = jnp.maximum(sl - WIN, 0)
    wpos = wstart[:, None] + jnp.arange(WIN, dtype=jnp.int32)[None, :]
    rows_w = ((bb[:, None, None] * L + wpos[:, None, :]) * HKV
              + jnp.arange(HKV, dtype=jnp.int32)[None, :, None])
    gidx = jnp.concatenate([rows_c.reshape(-1), rows_w.reshape(-1)])

    kg, vg = _sc_gather(k_cache.reshape(-1, D), v_cache.reshape(-1, D), gidx)

    o = _attn(seq_lens, q4.reshape(B, HKV, G, D),
              k_new.reshape(B, HKV, 1, D), v_new.reshape(B, HKV, 1, D),
              kn_w.reshape(1, D), qn_w.reshape(1, D),
              idxc.astype(jnp.float32).reshape(B * HKV, 1, NS),
              wch.reshape(B, HKV, G, TOPK), kg, vg)
    out = _outproj(o.reshape(B, DM), Wo, bo)
    return out[:, None, :]


# speedup vs baseline: 7.7985x; 1.0753x over previous
import functools

import jax
import jax.numpy as jnp
from jax import lax
from jax.experimental import pallas as pl
from jax.experimental.pallas import tpu as pltpu
from jax.experimental.pallas import tpu_sc as plsc

B = 32
L = 4096
HKV = 4
G = 4
HQ = 16
D = 128
DM = 2048
DKV = 512
TOPK = 8
CHUNK = 64
WIN = 128
NCH = 64
EPS = 1e-6
SCALE = 1.0 / (D ** 0.5)
HP = lax.Precision.HIGHEST

NS = G * TOPK
NTOK = NS * CHUNK
NROWS_C = B * HKV * NTOK
NROWS_W = B * HKV * WIN
NROWS = NROWS_C + NROWS_W

NW = 32
PW = NROWS // NW
CH_G = 128
NIT = PW // CH_G


def _prsqrt(x):
    y = lax.rsqrt(x)
    return y * (1.5 - 0.5 * x * y * y)


def _proj_body(h_ref, wq_ref, wkv_ref, bq_ref, bkv_ref, q_ref, kv_ref):
    h = h_ref[...]
    q_ref[...] = lax.dot_general(h, wq_ref[...], (((1,), (1,)), ((), ())),
                                 precision=HP) + bq_ref[...]
    kv_ref[...] = lax.dot_general(h, wkv_ref[...], (((1,), (1,)), ((), ())),
                                  precision=HP) + bkv_ref[...]


def _proj(h, Wq, bq, Wkv, bkv):
    return pl.pallas_call(
        _proj_body,
        out_shape=[jax.ShapeDtypeStruct((B, DM), jnp.float32),
                   jax.ShapeDtypeStruct((B, 2 * DKV), jnp.float32)],
    )(h, Wq, Wkv, bq.reshape(1, DM), bkv.reshape(1, 2 * DKV))


def _sc_gather(k2, v2, gidx):
    mesh = plsc.VectorSubcoreMesh(core_axis_name="c", subcore_axis_name="s")

    @functools.partial(
        pl.kernel,
        out_type=[jax.ShapeDtypeStruct((NROWS, D), jnp.float32),
                  jax.ShapeDtypeStruct((NROWS, D), jnp.float32)],
        mesh=mesh,
        scratch_types=[pltpu.VMEM((PW,), jnp.int32),
                       pltpu.VMEM((CH_G, D), jnp.float32),
                       pltpu.VMEM((CH_G, D), jnp.float32),
                       pltpu.VMEM((CH_G, D), jnp.float32),
                       pltpu.VMEM((CH_G, D), jnp.float32),
                       pltpu.SemaphoreType.DMA,
                       pltpu.SemaphoreType.DMA,
                       pltpu.SemaphoreType.DMA,
                       pltpu.SemaphoreType.DMA,
                       pltpu.SemaphoreType.DMA,
                       pltpu.SemaphoreType.DMA,
                       pltpu.SemaphoreType.DMA,
                       pltpu.SemaphoreType.DMA],
    )
    def kern(k_hbm, v_hbm, i_hbm, ok_hbm, ov_hbm, idx_v, kba, vba, kbb, vbb,
             gka, gva, gkb, gvb, wka, wva, wkb, wvb):
        wid = lax.axis_index("s") * 2 + lax.axis_index("c")
        base = wid * PW
        pltpu.sync_copy(i_hbm.at[pl.ds(base, PW)], idx_v)

        @pl.loop(0, NIT, step=2)
        def _(it):
            isla = idx_v.at[pl.ds(it * CH_G, CH_G)]
            hka = pltpu.async_copy(k_hbm.at[isla], kba, gka)
            hva = pltpu.async_copy(v_hbm.at[isla], vba, gva)
            islb = idx_v.at[pl.ds((it + 1) * CH_G, CH_G)]
            hkb = pltpu.async_copy(k_hbm.at[islb], kbb, gkb)
            hvb = pltpu.async_copy(v_hbm.at[islb], vbb, gvb)
            hka.wait()
            hva.wait()
            wa = pltpu.async_copy(kba, ok_hbm.at[pl.ds(base + it * CH_G, CH_G)], wka)
            wb = pltpu.async_copy(vba, ov_hbm.at[pl.ds(base + it * CH_G, CH_G)], wva)
            hkb.wait()
            hvb.wait()
            wc = pltpu.async_copy(kbb, ok_hbm.at[pl.ds(base + (it + 1) * CH_G, CH_G)], wkb)
            wd = pltpu.async_copy(vbb, ov_hbm.at[pl.ds(base + (it + 1) * CH_G, CH_G)], wvb)
            wa.wait()
            wb.wait()
            wc.wait()
            wd.wait()

    return kern(k2, v2, gidx)


def _attn_body(seq_ref, qn_ref, knew_ref, vnew_ref, knw_ref, qnw_ref,
               s_ref, gm_ref, pos_ref, wm_ref,
               kgc_ref, vgc_ref, kgw_ref, vgw_ref, o_ref):
    b = pl.program_id(0)
    sl = seq_ref[b] + 1
    slf = sl.astype(jnp.float32)
    qr = qn_ref[0, 0]
    qn = qnw_ref[...] * (qr * _prsqrt(jnp.mean(qr * qr, -1, keepdims=True)
                                      + EPS))
    qk = qn * knw_ref[...]
    knew = knew_ref[0, 0]
    vnew = vnew_ref[0, 0]
    lnew = lax.dot_general(qk, knew, (((1,), (1,)), ((), ())),
                           precision=HP)
    ssnew = jnp.sum(knew * knew, axis=1, keepdims=True)

    kc = kgc_ref[...]
    vc = vgc_ref[...]
    S = s_ref[...]
    gm = gm_ref[...]
    pos = pos_ref[0]
    valid = (pos < slf).astype(jnp.float32)
    isnew = (pos == slf - 1.0).astype(jnp.float32)
    l4 = lax.dot_general(qk, kc, (((1,), (1,)), ((), ())),
                         precision=HP)
    lc = jnp.sum(l4 * gm, axis=0, keepdims=True)
    lnewc = jnp.sum(lnew * gm, axis=0, keepdims=True)
    lc = lc + isnew * (lnewc - lc)
    ones = jnp.ones((1, D), dtype=jnp.float32)
    ss = lax.dot_general(ones, kc * kc, (((1,), (1,)), ((), ())),
                         precision=HP)
    ss = ss + isnew * (ssnew - ss)
    rinv = _prsqrt(ss * (1.0 / D) + EPS)
    ec = jnp.exp(lc * rinv * SCALE) * valid
    d32 = lax.dot_general(ec, S, (((1,), (1,)), ((), ())), precision=HP)
    dlane = lax.dot_general(d32, S, (((1,), (0,)), ((), ())), precision=HP)
    attn = ec / (dlane + 1e-30)
    tcol = jnp.transpose(isnew, (1, 0))
    vcp = vc + tcol * (vnew - vc)
    o32 = lax.dot_general(S * attn, vcp, (((1,), (0,)), ((), ())))
    o_hsa = lax.dot_general(wm_ref[0, 0], o32, (((1,), (0,)), ((), ())),
                            precision=HP)

    kw_ = kgw_ref[...]
    vw_ = vgw_ref[...]
    wlane = lax.broadcasted_iota(jnp.int32, (1, WIN), 1)
    wstart = jnp.maximum(sl - WIN, 0)
    wpos = wlane + wstart
    wvalid = (wpos < sl).astype(jnp.float32)
    wisnew = (wpos == sl - 1).astype(jnp.float32)
    lw = lax.dot_general(qk, kw_, (((1,), (1,)), ((), ())),
                         precision=HP)
    lw = lw + wisnew * (lnew - lw)
    ssw = lax.dot_general(ones, kw_ * kw_, (((1,), (1,)), ((), ())),
                          precision=HP)
    ssw = ssw + wisnew * (ssnew - ssw)
    rinvw = _prsqrt(ssw * (1.0 / D) + EPS)
    ew = jnp.exp(lw * rinvw * SCALE) * wvalid
    dw = jnp.sum(ew, axis=1, keepdims=True)
    aw = ew / dw
    twin = jnp.transpose(wisnew, (1, 0))
    vwp = vw_ + twin * (vnew - vw_)
    o_swa = lax.dot_general(aw, vwp, (((1,), (0,)), ((), ())), precision=HP)
    o_ref[0, 0] = o_hsa + o_swa


def _attn(seq_lens, qn4, knew4, vnew4, kn_w, qn_w, smat, gmat, posf, wmat,
          kg, vg):
    return pl.pallas_call(
        _attn_body,
        grid=(B, HKV),
        in_specs=[
            pl.BlockSpec(memory_space=pltpu.SMEM),
            pl.BlockSpec((1, 1, G, D), lambda b, k: (b, k, 0, 0)),
            pl.BlockSpec((1, 1, 1, D), lambda b, k: (b, k, 0, 0)),
            pl.BlockSpec((1, 1, 1, D), lambda b, k: (b, k, 0, 0)),
            pl.BlockSpec((1, D), lambda b, k: (0, 0)),
            pl.BlockSpec((1, D), lambda b, k: (0, 0)),
            pl.BlockSpec((NS, NTOK), lambda b, k: (0, 0)),
            pl.BlockSpec((G, NTOK), lambda b, k: (0, 0)),
            pl.BlockSpec((1, 1, NTOK), lambda b, k: (b * HKV + k, 0, 0)),
            pl.BlockSpec((1, 1, G, NS), lambda b, k: (b, k, 0, 0)),
            pl.BlockSpec((NTOK, D), lambda b, k: (b * HKV + k, 0)),
            pl.BlockSpec((NTOK, D), lambda b, k: (b * HKV + k, 0)),
            pl.BlockSpec((WIN, D), lambda b, k: (NROWS_C // WIN + b * HKV + k, 0)),
            pl.BlockSpec((WIN, D), lambda b, k: (NROWS_C // WIN + b * HKV + k, 0)),
        ],
        out_specs=pl.BlockSpec((1, 1, G, D), lambda b, k: (b, k, 0, 0)),
        out_shape=jax.ShapeDtypeStruct((B, HKV, G, D), jnp.float32),
    )(seq_lens, qn4, knew4, vnew4, kn_w, qn_w, smat, gmat, posf, wmat,
      kg, vg, kg, vg)


def _out_body(o_ref, wo_ref, bo_ref, out_ref):
    out_ref[...] = lax.dot_general(o_ref[...], wo_ref[...],
                                   (((1,), (1,)), ((), ())),
                                   precision=HP) + bo_ref[...]


def _outproj(o2, Wo, bo):
    return pl.pallas_call(
        _out_body,
        out_shape=jax.ShapeDtypeStruct((B, DM), jnp.float32),
    )(o2, Wo, bo.reshape(1, DM))


def kernel(hidden_states, k_cache, v_cache, seq_lens, Wq, bq, Wkv, bkv,
           Wo, bo, qn_w, kn_w, ln_w):
    h = hidden_states[:, 0, :]
    q_r, kv = _proj(h, Wq, bq, Wkv, bkv)
    q4 = q_r.reshape(B, HQ, D)
    k_new = kv[:, :DKV].reshape(B, HKV, D)
    v_new = kv[:, DKV:].reshape(B, HKV, D)

    def _rms(x, w):
        xf = x.astype(jnp.float32)
        return w * (xf * lax.rsqrt(jnp.mean(xf * xf, -1, keepdims=True) + EPS))

    qs = _rms((h @ Wq.T + bq).reshape(B, HKV, G, D), qn_w)
    kvs = h @ Wkv.T + bkv
    kns = kvs[:, :DKV].reshape(B, HKV, D)
    js = jnp.arange(NCH)
    lmks = k_cache[:, ::CHUNK]
    lmks = jnp.where(((js * CHUNK)[None, :] == seq_lens[:, None])[..., None, None],
                     kns[:, None], lmks)
    lmks = _rms(lmks, ln_w)
    scores = jnp.einsum('bkgd,bckd->bkgc', qs, lmks) * SCALE
    scores = jnp.where(((js * CHUNK)[None, :] < (seq_lens + 1)[:, None])
                       [:, None, None, :], scores, -1e9)
    vals, idx_s = jax.lax.top_k(scores, TOPK)
    wsel = jax.nn.softmax(vals, axis=-1)
    wch = wsel.reshape(B, HQ, TOPK)
    idxc = idx_s.astype(jnp.int32).reshape(B, HQ, TOPK)

    bb = jnp.arange(B, dtype=jnp.int32)
    pos = idxc[..., None] * CHUNK + jnp.arange(CHUNK, dtype=jnp.int32)
    kofhq = (jnp.arange(HQ, dtype=jnp.int32) // G)[None, :, None, None]
    rows_c = (bb[:, None, None, None] * L + pos) * HKV + kofhq
    sl = seq_lens + 1
    wstart = jnp.maximum(sl - WIN, 0)
    wpos = wstart[:, None] + jnp.arange(WIN, dtype=jnp.int32)[None, :]
    rows_w = ((bb[:, None, None] * L + wpos[:, None, :]) * HKV
              + jnp.arange(HKV, dtype=jnp.int32)[None, :, None])
    gidx = jnp.concatenate([rows_c.reshape(-1), rows_w.reshape(-1)])

    kg, vg = _sc_gather(k_cache.reshape(-1, D), v_cache.reshape(-1, D), gidx)

    lanes = jnp.arange(NTOK, dtype=jnp.int32)
    smat = (jnp.arange(NS, dtype=jnp.int32)[:, None]
            == lanes[None, :] // CHUNK).astype(jnp.float32)
    gmat = (jnp.arange(G, dtype=jnp.int32)[:, None]
            == lanes[None, :] // (TOPK * CHUNK)).astype(jnp.float32)
    posf = pos.reshape(B * HKV, 1, NTOK).astype(jnp.float32)
    wmat = (wch.reshape(B, HKV, G, 1, TOPK)
            * jnp.eye(G, dtype=jnp.float32)[None, None, :, :, None]
            ).reshape(B, HKV, G, NS)

    o = _attn(seq_lens, q4.reshape(B, HKV, G, D),
              k_new.reshape(B, HKV, 1, D), v_new.reshape(B, HKV, 1, D),
              kn_w.reshape(1, D), qn_w.reshape(1, D),
              smat, gmat, posf, wmat, kg, vg)
    out = _outproj(o.reshape(B, DM), Wo, bo)
    return out[:, None, :]


# speedup vs baseline: 8.7007x; 1.1157x over previous
import functools

import jax
import jax.numpy as jnp
from jax import lax
from jax.experimental import pallas as pl
from jax.experimental.pallas import tpu as pltpu
from jax.experimental.pallas import tpu_sc as plsc

B = 32
L = 4096
HKV = 4
G = 4
HQ = 16
D = 128
DM = 2048
DKV = 512
TOPK = 8
CHUNK = 64
WIN = 128
NCH = 64
EPS = 1e-6
SCALE = 1.0 / (D ** 0.5)
HP = lax.Precision.HIGHEST

NS = G * TOPK
NTOK = NS * CHUNK
NROWS_C = B * HKV * NTOK
NROWS_W = B * HKV * WIN
NROWS = NROWS_C + NROWS_W

NW = 32
PW = NROWS // NW
CH_G = 128
NIT = PW // CH_G


def _prsqrt(x):
    y = lax.rsqrt(x)
    return y * (1.5 - 0.5 * x * y * y)


def _proj_body(h_ref, wq_ref, wkv_ref, bq_ref, bkv_ref, q_ref, kv_ref):
    h = h_ref[...]
    q_ref[...] = lax.dot_general(h, wq_ref[...], (((1,), (1,)), ((), ())),
                                 precision=HP) + bq_ref[...]
    kv_ref[...] = lax.dot_general(h, wkv_ref[...], (((1,), (1,)), ((), ())),
                                  precision=HP) + bkv_ref[...]


def _proj(h, Wq, bq, Wkv, bkv):
    return pl.pallas_call(
        _proj_body,
        out_shape=[jax.ShapeDtypeStruct((B, DM), jnp.float32),
                   jax.ShapeDtypeStruct((B, 2 * DKV), jnp.float32)],
    )(h, Wq, Wkv, bq.reshape(1, DM), bkv.reshape(1, 2 * DKV))


def _sc_gather(k2, v2, gidx):
    nrows = gidx.shape[0]
    pw = nrows // NW
    nit = pw // CH_G
    mesh = plsc.VectorSubcoreMesh(core_axis_name="c", subcore_axis_name="s")

    @functools.partial(
        pl.kernel,
        out_type=[jax.ShapeDtypeStruct((nrows, D), jnp.float32),
                  jax.ShapeDtypeStruct((nrows, D), jnp.float32)],
        mesh=mesh,
        scratch_types=[pltpu.VMEM((pw,), jnp.int32),
                       pltpu.VMEM((CH_G, D), jnp.float32),
                       pltpu.VMEM((CH_G, D), jnp.float32),
                       pltpu.VMEM((CH_G, D), jnp.float32),
                       pltpu.VMEM((CH_G, D), jnp.float32),
                       pltpu.SemaphoreType.DMA,
                       pltpu.SemaphoreType.DMA,
                       pltpu.SemaphoreType.DMA,
                       pltpu.SemaphoreType.DMA,
                       pltpu.SemaphoreType.DMA,
                       pltpu.SemaphoreType.DMA,
                       pltpu.SemaphoreType.DMA,
                       pltpu.SemaphoreType.DMA],
    )
    def kern(k_hbm, v_hbm, i_hbm, ok_hbm, ov_hbm, idx_v, kba, vba, kbb, vbb,
             gka, gva, gkb, gvb, wka, wva, wkb, wvb):
        wid = lax.axis_index("s") * 2 + lax.axis_index("c")
        base = wid * pw
        pltpu.sync_copy(i_hbm.at[pl.ds(base, pw)], idx_v)

        @pl.loop(0, nit, step=2)
        def _(it):
            isla = idx_v.at[pl.ds(it * CH_G, CH_G)]
            hka = pltpu.async_copy(k_hbm.at[isla], kba, gka)
            hva = pltpu.async_copy(v_hbm.at[isla], vba, gva)
            islb = idx_v.at[pl.ds((it + 1) * CH_G, CH_G)]
            hkb = pltpu.async_copy(k_hbm.at[islb], kbb, gkb)
            hvb = pltpu.async_copy(v_hbm.at[islb], vbb, gvb)
            hka.wait()
            hva.wait()
            wa = pltpu.async_copy(kba, ok_hbm.at[pl.ds(base + it * CH_G, CH_G)], wka)
            wb = pltpu.async_copy(vba, ov_hbm.at[pl.ds(base + it * CH_G, CH_G)], wva)
            hkb.wait()
            hvb.wait()
            wc = pltpu.async_copy(kbb, ok_hbm.at[pl.ds(base + (it + 1) * CH_G, CH_G)], wkb)
            wd = pltpu.async_copy(vbb, ov_hbm.at[pl.ds(base + (it + 1) * CH_G, CH_G)], wvb)
            wa.wait()
            wb.wait()
            wc.wait()
            wd.wait()

    return kern(k2, v2, gidx)


def _attn_body(seq_ref, qn_ref, knew_ref, vnew_ref, knw_ref, qnw_ref,
               s_ref, gm_ref, pos_ref, wm_ref,
               kgc_ref, vgc_ref, kgw_ref, vgw_ref, o_ref):
    b = pl.program_id(0)
    sl = seq_ref[b] + 1
    slf = sl.astype(jnp.float32)
    qr = qn_ref[0, 0]
    qn = qnw_ref[...] * (qr * _prsqrt(jnp.mean(qr * qr, -1, keepdims=True)
                                      + EPS))
    qk = qn * knw_ref[...]
    knew = knew_ref[0, 0]
    vnew = vnew_ref[0, 0]
    lnew = lax.dot_general(qk, knew, (((1,), (1,)), ((), ())),
                           precision=HP)
    ssnew = jnp.sum(knew * knew, axis=1, keepdims=True)

    kc = kgc_ref[...]
    vc = vgc_ref[...]
    S = s_ref[...]
    gm = gm_ref[...]
    pos = pos_ref[0]
    valid = (pos < slf).astype(jnp.float32)
    isnew = (pos == slf - 1.0).astype(jnp.float32)
    l4 = lax.dot_general(qk, kc, (((1,), (1,)), ((), ())),
                         precision=HP)
    lc = jnp.sum(l4 * gm, axis=0, keepdims=True)
    lnewc = jnp.sum(lnew * gm, axis=0, keepdims=True)
    lc = lc + isnew * (lnewc - lc)
    ones = jnp.ones((1, D), dtype=jnp.float32)
    ss = lax.dot_general(ones, kc * kc, (((1,), (1,)), ((), ())),
                         precision=HP)
    ss = ss + isnew * (ssnew - ss)
    rinv = _prsqrt(ss * (1.0 / D) + EPS)
    ec = jnp.exp(lc * rinv * SCALE) * valid
    d32 = lax.dot_general(ec, S, (((1,), (1,)), ((), ())), precision=HP)
    dlane = lax.dot_general(d32, S, (((1,), (0,)), ((), ())), precision=HP)
    attn = ec / (dlane + 1e-30)
    tcol = jnp.transpose(isnew, (1, 0))
    vcp = vc + tcol * (vnew - vc)
    o32 = lax.dot_general(S * attn, vcp, (((1,), (0,)), ((), ())))
    o_hsa = lax.dot_general(wm_ref[0, 0], o32, (((1,), (0,)), ((), ())),
                            precision=HP)

    kw_ = kgw_ref[...]
    vw_ = vgw_ref[...]
    wlane = lax.broadcasted_iota(jnp.int32, (1, WIN), 1)
    wstart = jnp.maximum(sl - WIN, 0)
    wpos = wlane + wstart
    wvalid = (wpos < sl).astype(jnp.float32)
    wisnew = (wpos == sl - 1).astype(jnp.float32)
    lw = lax.dot_general(qk, kw_, (((1,), (1,)), ((), ())),
                         precision=HP)
    lw = lw + wisnew * (lnew - lw)
    ssw = lax.dot_general(ones, kw_ * kw_, (((1,), (1,)), ((), ())),
                          precision=HP)
    ssw = ssw + wisnew * (ssnew - ssw)
    rinvw = _prsqrt(ssw * (1.0 / D) + EPS)
    ew = jnp.exp(lw * rinvw * SCALE) * wvalid
    dw = jnp.sum(ew, axis=1, keepdims=True)
    aw = ew / dw
    twin = jnp.transpose(wisnew, (1, 0))
    vwp = vw_ + twin * (vnew - vw_)
    o_swa = lax.dot_general(aw, vwp, (((1,), (0,)), ((), ())), precision=HP)
    o_ref[0, 0] = o_hsa + o_swa


def _attn(seq_lens, qn4, knew4, vnew4, kn_w, qn_w, smat, gmat, posf, wmat,
          kg, vg):
    bh = qn4.shape[0]
    nrc = bh * HKV * NTOK
    return pl.pallas_call(
        _attn_body,
        grid=(bh, HKV),
        in_specs=[
            pl.BlockSpec(memory_space=pltpu.SMEM),
            pl.BlockSpec((1, 1, G, D), lambda b, k: (b, k, 0, 0)),
            pl.BlockSpec((1, 1, 1, D), lambda b, k: (b, k, 0, 0)),
            pl.BlockSpec((1, 1, 1, D), lambda b, k: (b, k, 0, 0)),
            pl.BlockSpec((1, D), lambda b, k: (0, 0)),
            pl.BlockSpec((1, D), lambda b, k: (0, 0)),
            pl.BlockSpec((NS, NTOK), lambda b, k: (0, 0)),
            pl.BlockSpec((G, NTOK), lambda b, k: (0, 0)),
            pl.BlockSpec((1, 1, NTOK), lambda b, k: (b * HKV + k, 0, 0)),
            pl.BlockSpec((1, 1, G, NS), lambda b, k: (b, k, 0, 0)),
            pl.BlockSpec((NTOK, D), lambda b, k: (b * HKV + k, 0)),
            pl.BlockSpec((NTOK, D), lambda b, k: (b * HKV + k, 0)),
            pl.BlockSpec((WIN, D), lambda b, k: (nrc // WIN + b * HKV + k, 0)),
            pl.BlockSpec((WIN, D), lambda b, k: (nrc // WIN + b * HKV + k, 0)),
        ],
        out_specs=pl.BlockSpec((1, 1, G, D), lambda b, k: (b, k, 0, 0)),
        out_shape=jax.ShapeDtypeStruct((bh, HKV, G, D), jnp.float32),
    )(seq_lens, qn4, knew4, vnew4, kn_w, qn_w, smat, gmat, posf, wmat,
      kg, vg, kg, vg)


def _out_body(o_ref, wo_ref, bo_ref, out_ref):
    out_ref[...] = lax.dot_general(o_ref[...], wo_ref[...],
                                   (((1,), (1,)), ((), ())),
                                   precision=HP) + bo_ref[...]


def _outproj(o2, Wo, bo):
    return pl.pallas_call(
        _out_body,
        out_shape=jax.ShapeDtypeStruct((B, DM), jnp.float32),
    )(o2, Wo, bo.reshape(1, DM))


def kernel(hidden_states, k_cache, v_cache, seq_lens, Wq, bq, Wkv, bkv,
           Wo, bo, qn_w, kn_w, ln_w):
    h = hidden_states[:, 0, :]
    q_r, kv = _proj(h, Wq, bq, Wkv, bkv)
    q4 = q_r.reshape(B, HQ, D)
    k_new = kv[:, :DKV].reshape(B, HKV, D)
    v_new = kv[:, DKV:].reshape(B, HKV, D)

    def _rms(x, w):
        xf = x.astype(jnp.float32)
        return w * (xf * lax.rsqrt(jnp.mean(xf * xf, -1, keepdims=True) + EPS))

    qs = _rms((h @ Wq.T + bq).reshape(B, HKV, G, D), qn_w)
    kvs = h @ Wkv.T + bkv
    kns = kvs[:, :DKV].reshape(B, HKV, D)
    js = jnp.arange(NCH)
    lmks = k_cache[:, ::CHUNK]
    lmks = jnp.where(((js * CHUNK)[None, :] == seq_lens[:, None])[..., None, None],
                     kns[:, None], lmks)
    lmks = _rms(lmks, ln_w)
    scores = jnp.einsum('bkgd,bckd->bkgc', qs, lmks) * SCALE
    scores = jnp.where(((js * CHUNK)[None, :] < (seq_lens + 1)[:, None])
                       [:, None, None, :], scores, -1e9)
    vals, idx_s = jax.lax.top_k(scores, TOPK)
    wsel = jax.nn.softmax(vals, axis=-1)
    wch = wsel.reshape(B, HQ, TOPK)
    idxc = idx_s.astype(jnp.int32).reshape(B, HQ, TOPK)

    bb = jnp.arange(B, dtype=jnp.int32)
    pos = idxc[..., None] * CHUNK + jnp.arange(CHUNK, dtype=jnp.int32)
    kofhq = (jnp.arange(HQ, dtype=jnp.int32) // G)[None, :, None, None]
    rows_c = (bb[:, None, None, None] * L + pos) * HKV + kofhq
    sl = seq_lens + 1
    wstart = jnp.maximum(sl - WIN, 0)
    wpos = wstart[:, None] + jnp.arange(WIN, dtype=jnp.int32)[None, :]
    rows_w = ((bb[:, None, None] * L + wpos[:, None, :]) * HKV
              + jnp.arange(HKV, dtype=jnp.int32)[None, :, None])

    lanes = jnp.arange(NTOK, dtype=jnp.int32)
    smat = (jnp.arange(NS, dtype=jnp.int32)[:, None]
            == lanes[None, :] // CHUNK).astype(jnp.float32)
    gmat = (jnp.arange(G, dtype=jnp.int32)[:, None]
            == lanes[None, :] // (TOPK * CHUNK)).astype(jnp.float32)
    posf = pos.reshape(B * HKV, 1, NTOK).astype(jnp.float32)
    wmat = (wch.reshape(B, HKV, G, 1, TOPK)
            * jnp.eye(G, dtype=jnp.float32)[None, None, :, :, None]
            ).reshape(B, HKV, G, NS)

    k2 = k_cache.reshape(-1, D)
    v2 = v_cache.reshape(-1, D)
    q44 = q4.reshape(B, HKV, G, D)
    kn4 = k_new.reshape(B, HKV, 1, D)
    vn4 = v_new.reshape(B, HKV, 1, D)
    knw = kn_w.reshape(1, D)
    qnw = qn_w.reshape(1, D)
    BH = B // 2
    outs = []
    for h in range(2):
        s0, s1 = h * BH, (h + 1) * BH
        gidx_h = jnp.concatenate([rows_c[s0:s1].reshape(-1),
                                  rows_w[s0:s1].reshape(-1)])
        kg, vg = _sc_gather(k2, v2, gidx_h)
        outs.append(_attn(seq_lens[s0:s1], q44[s0:s1], kn4[s0:s1], vn4[s0:s1],
                          knw, qnw, smat, gmat,
                          posf[s0 * HKV:s1 * HKV], wmat[s0:s1], kg, vg))
    o = jnp.concatenate(outs, axis=0)
    out = _outproj(o.reshape(B, DM), Wo, bo)
    return out[:, None, :]


# speedup vs baseline: 9.2354x; 1.0615x over previous
import functools

import jax
import jax.numpy as jnp
from jax import lax
from jax.experimental import pallas as pl
from jax.experimental.pallas import tpu as pltpu
from jax.experimental.pallas import tpu_sc as plsc

B = 32
L = 4096
HKV = 4
G = 4
HQ = 16
D = 128
DM = 2048
DKV = 512
TOPK = 8
CHUNK = 64
WIN = 128
NCH = 64
EPS = 1e-6
SCALE = 1.0 / (D ** 0.5)
HP = lax.Precision.HIGHEST

NS = G * TOPK
NTOK = NS * CHUNK
NROWS_C = B * HKV * NTOK
NROWS_W = B * HKV * WIN
NROWS = NROWS_C + NROWS_W

NW = 32
PW = NROWS // NW
CH_G = 128
NIT = PW // CH_G


def _prsqrt(x):
    y = lax.rsqrt(x)
    return y * (1.5 - 0.5 * x * y * y)


def _proj_body(h_ref, wq_ref, wkv_ref, bq_ref, bkv_ref, q_ref, kv_ref):
    h = h_ref[...]
    q_ref[...] = lax.dot_general(h, wq_ref[...], (((1,), (1,)), ((), ())),
                                 precision=HP) + bq_ref[...]
    kv_ref[...] = lax.dot_general(h, wkv_ref[...], (((1,), (1,)), ((), ())),
                                  precision=HP) + bkv_ref[...]


def _proj(h, Wq, bq, Wkv, bkv):
    return pl.pallas_call(
        _proj_body,
        out_shape=[jax.ShapeDtypeStruct((B, DM), jnp.float32),
                   jax.ShapeDtypeStruct((B, 2 * DKV), jnp.float32)],
    )(h, Wq, Wkv, bq.reshape(1, DM), bkv.reshape(1, 2 * DKV))


def _sc_gather(k2, v2, gidx):
    nrows = gidx.shape[0]
    pw = nrows // NW
    ch = CH_G if (pw // CH_G) % 2 == 0 else CH_G // 2
    nit = pw // ch
    mesh = plsc.VectorSubcoreMesh(core_axis_name="c", subcore_axis_name="s")

    @functools.partial(
        pl.kernel,
        out_type=[jax.ShapeDtypeStruct((nrows, D), jnp.float32),
                  jax.ShapeDtypeStruct((nrows, D), jnp.float32)],
        mesh=mesh,
        scratch_types=[pltpu.VMEM((pw,), jnp.int32),
                       pltpu.VMEM((ch, D), jnp.float32),
                       pltpu.VMEM((ch, D), jnp.float32),
                       pltpu.VMEM((ch, D), jnp.float32),
                       pltpu.VMEM((ch, D), jnp.float32),
                       pltpu.SemaphoreType.DMA,
                       pltpu.SemaphoreType.DMA,
                       pltpu.SemaphoreType.DMA,
                       pltpu.SemaphoreType.DMA,
                       pltpu.SemaphoreType.DMA,
                       pltpu.SemaphoreType.DMA,
                       pltpu.SemaphoreType.DMA,
                       pltpu.SemaphoreType.DMA],
    )
    def kern(k_hbm, v_hbm, i_hbm, ok_hbm, ov_hbm, idx_v, kba, vba, kbb, vbb,
             gka, gva, gkb, gvb, wka, wva, wkb, wvb):
        wid = lax.axis_index("s") * 2 + lax.axis_index("c")
        base = wid * pw
        pltpu.sync_copy(i_hbm.at[pl.ds(base, pw)], idx_v)

        @pl.loop(0, nit, step=2)
        def _(it):
            isla = idx_v.at[pl.ds(it * ch, ch)]
            hka = pltpu.async_copy(k_hbm.at[isla], kba, gka)
            hva = pltpu.async_copy(v_hbm.at[isla], vba, gva)
            islb = idx_v.at[pl.ds((it + 1) * ch, ch)]
            hkb = pltpu.async_copy(k_hbm.at[islb], kbb, gkb)
            hvb = pltpu.async_copy(v_hbm.at[islb], vbb, gvb)
            hka.wait()
            hva.wait()
            wa = pltpu.async_copy(kba, ok_hbm.at[pl.ds(base + it * ch, ch)], wka)
            wb = pltpu.async_copy(vba, ov_hbm.at[pl.ds(base + it * ch, ch)], wva)
            hkb.wait()
            hvb.wait()
            wc = pltpu.async_copy(kbb, ok_hbm.at[pl.ds(base + (it + 1) * ch, ch)], wkb)
            wd = pltpu.async_copy(vbb, ov_hbm.at[pl.ds(base + (it + 1) * ch, ch)], wvb)
            wa.wait()
            wb.wait()
            wc.wait()
            wd.wait()

    return kern(k2, v2, gidx)


def _attn_body(seq_ref, qn_ref, knew_ref, vnew_ref, knw_ref, qnw_ref,
               s_ref, gm_ref, pos_ref, wm_ref,
               kgc_ref, vgc_ref, kgw_ref, vgw_ref, o_ref):
    b = pl.program_id(0)
    sl = seq_ref[b] + 1
    slf = sl.astype(jnp.float32)
    qr = qn_ref[0, 0]
    qn = qnw_ref[...] * (qr * _prsqrt(jnp.mean(qr * qr, -1, keepdims=True)
                                      + EPS))
    qk = qn * knw_ref[...]
    knew = knew_ref[0, 0]
    vnew = vnew_ref[0, 0]
    lnew = lax.dot_general(qk, knew, (((1,), (1,)), ((), ())),
                           precision=HP)
    ssnew = jnp.sum(knew * knew, axis=1, keepdims=True)

    kc = kgc_ref[...]
    vc = vgc_ref[...]
    S = s_ref[...]
    gm = gm_ref[...]
    pos = pos_ref[0]
    valid = (pos < slf).astype(jnp.float32)
    isnew = (pos == slf - 1.0).astype(jnp.float32)
    l4 = lax.dot_general(qk, kc, (((1,), (1,)), ((), ())),
                         precision=HP)
    lc = jnp.sum(l4 * gm, axis=0, keepdims=True)
    lnewc = jnp.sum(lnew * gm, axis=0, keepdims=True)
    lc = lc + isnew * (lnewc - lc)
    ones = jnp.ones((1, D), dtype=jnp.float32)
    ss = lax.dot_general(ones, kc * kc, (((1,), (1,)), ((), ())),
                         precision=HP)
    ss = ss + isnew * (ssnew - ss)
    rinv = _prsqrt(ss * (1.0 / D) + EPS)
    ec = jnp.exp(lc * rinv * SCALE) * valid
    d32 = lax.dot_general(ec, S, (((1,), (1,)), ((), ())), precision=HP)
    dlane = lax.dot_general(d32, S, (((1,), (0,)), ((), ())), precision=HP)
    attn = ec / (dlane + 1e-30)
    tcol = jnp.transpose(isnew, (1, 0))
    vcp = vc + tcol * (vnew - vc)
    o32 = lax.dot_general(S * attn, vcp, (((1,), (0,)), ((), ())))
    o_hsa = lax.dot_general(wm_ref[0, 0], o32, (((1,), (0,)), ((), ())),
                            precision=HP)

    kw_ = kgw_ref[...]
    vw_ = vgw_ref[...]
    wlane = lax.broadcasted_iota(jnp.int32, (1, WIN), 1)
    wstart = jnp.maximum(sl - WIN, 0)
    wpos = wlane + wstart
    wvalid = (wpos < sl).astype(jnp.float32)
    wisnew = (wpos == sl - 1).astype(jnp.float32)
    lw = lax.dot_general(qk, kw_, (((1,), (1,)), ((), ())),
                         precision=HP)
    lw = lw + wisnew * (lnew - lw)
    ssw = lax.dot_general(ones, kw_ * kw_, (((1,), (1,)), ((), ())),
                          precision=HP)
    ssw = ssw + wisnew * (ssnew - ssw)
    rinvw = _prsqrt(ssw * (1.0 / D) + EPS)
    ew = jnp.exp(lw * rinvw * SCALE) * wvalid
    dw = jnp.sum(ew, axis=1, keepdims=True)
    aw = ew / dw
    twin = jnp.transpose(wisnew, (1, 0))
    vwp = vw_ + twin * (vnew - vw_)
    o_swa = lax.dot_general(aw, vwp, (((1,), (0,)), ((), ())), precision=HP)
    o_ref[0, 0] = o_hsa + o_swa


def _attn(seq_lens, qn4, knew4, vnew4, kn_w, qn_w, smat, gmat, posf, wmat,
          kg, vg):
    bh = qn4.shape[0]
    nrc = bh * HKV * NTOK
    return pl.pallas_call(
        _attn_body,
        grid=(bh, HKV),
        in_specs=[
            pl.BlockSpec(memory_space=pltpu.SMEM),
            pl.BlockSpec((1, 1, G, D), lambda b, k: (b, k, 0, 0)),
            pl.BlockSpec((1, 1, 1, D), lambda b, k: (b, k, 0, 0)),
            pl.BlockSpec((1, 1, 1, D), lambda b, k: (b, k, 0, 0)),
            pl.BlockSpec((1, D), lambda b, k: (0, 0)),
            pl.BlockSpec((1, D), lambda b, k: (0, 0)),
            pl.BlockSpec((NS, NTOK), lambda b, k: (0, 0)),
            pl.BlockSpec((G, NTOK), lambda b, k: (0, 0)),
            pl.BlockSpec((1, 1, NTOK), lambda b, k: (b * HKV + k, 0, 0)),
            pl.BlockSpec((1, 1, G, NS), lambda b, k: (b, k, 0, 0)),
            pl.BlockSpec((NTOK, D), lambda b, k: (b * HKV + k, 0)),
            pl.BlockSpec((NTOK, D), lambda b, k: (b * HKV + k, 0)),
            pl.BlockSpec((WIN, D), lambda b, k: (nrc // WIN + b * HKV + k, 0)),
            pl.BlockSpec((WIN, D), lambda b, k: (nrc // WIN + b * HKV + k, 0)),
        ],
        out_specs=pl.BlockSpec((1, 1, G, D), lambda b, k: (b, k, 0, 0)),
        out_shape=jax.ShapeDtypeStruct((bh, HKV, G, D), jnp.float32),
    )(seq_lens, qn4, knew4, vnew4, kn_w, qn_w, smat, gmat, posf, wmat,
      kg, vg, kg, vg)


def _out_body(o_ref, wo_ref, bo_ref, out_ref):
    out_ref[...] = lax.dot_general(o_ref[...], wo_ref[...],
                                   (((1,), (1,)), ((), ())),
                                   precision=HP) + bo_ref[...]


def _outproj(o2, Wo, bo):
    return pl.pallas_call(
        _out_body,
        out_shape=jax.ShapeDtypeStruct((B, DM), jnp.float32),
    )(o2, Wo, bo.reshape(1, DM))


def kernel(hidden_states, k_cache, v_cache, seq_lens, Wq, bq, Wkv, bkv,
           Wo, bo, qn_w, kn_w, ln_w):
    h = hidden_states[:, 0, :]
    q_r, kv = _proj(h, Wq, bq, Wkv, bkv)
    q4 = q_r.reshape(B, HQ, D)
    k_new = kv[:, :DKV].reshape(B, HKV, D)
    v_new = kv[:, DKV:].reshape(B, HKV, D)

    def _rms(x, w):
        xf = x.astype(jnp.float32)
        return w * (xf * lax.rsqrt(jnp.mean(xf * xf, -1, keepdims=True) + EPS))

    qs = _rms((h @ Wq.T + bq).reshape(B, HKV, G, D), qn_w)
    kvs = h @ Wkv.T + bkv
    kns = kvs[:, :DKV].reshape(B, HKV, D)
    js = jnp.arange(NCH)
    lmks = k_cache[:, ::CHUNK]
    lmks = jnp.where(((js * CHUNK)[None, :] == seq_lens[:, None])[..., None, None],
                     kns[:, None], lmks)
    lmks = _rms(lmks, ln_w)
    scores = jnp.einsum('bkgd,bckd->bkgc', qs, lmks) * SCALE
    scores = jnp.where(((js * CHUNK)[None, :] < (seq_lens + 1)[:, None])
                       [:, None, None, :], scores, -1e9)
    vals, idx_s = jax.lax.top_k(scores, TOPK)
    wsel = jax.nn.softmax(vals, axis=-1)
    wch = wsel.reshape(B, HQ, TOPK)
    idxc = idx_s.astype(jnp.int32).reshape(B, HQ, TOPK)

    bb = jnp.arange(B, dtype=jnp.int32)
    pos = idxc[..., None] * CHUNK + jnp.arange(CHUNK, dtype=jnp.int32)
    kofhq = (jnp.arange(HQ, dtype=jnp.int32) // G)[None, :, None, None]
    rows_c = (bb[:, None, None, None] * L + pos) * HKV + kofhq
    sl = seq_lens + 1
    wstart = jnp.maximum(sl - WIN, 0)
    wpos = wstart[:, None] + jnp.arange(WIN, dtype=jnp.int32)[None, :]
    rows_w = ((bb[:, None, None] * L + wpos[:, None, :]) * HKV
              + jnp.arange(HKV, dtype=jnp.int32)[None, :, None])

    lanes = jnp.arange(NTOK, dtype=jnp.int32)
    smat = (jnp.arange(NS, dtype=jnp.int32)[:, None]
            == lanes[None, :] // CHUNK).astype(jnp.float32)
    gmat = (jnp.arange(G, dtype=jnp.int32)[:, None]
            == lanes[None, :] // (TOPK * CHUNK)).astype(jnp.float32)
    posf = pos.reshape(B * HKV, 1, NTOK).astype(jnp.float32)
    wmat = (wch.reshape(B, HKV, G, 1, TOPK)
            * jnp.eye(G, dtype=jnp.float32)[None, None, :, :, None]
            ).reshape(B, HKV, G, NS)

    k2 = k_cache.reshape(-1, D)
    v2 = v_cache.reshape(-1, D)
    q44 = q4.reshape(B, HKV, G, D)
    kn4 = k_new.reshape(B, HKV, 1, D)
    vn4 = v_new.reshape(B, HKV, 1, D)
    knw = kn_w.reshape(1, D)
    qnw = qn_w.reshape(1, D)
    BH = B // 4
    outs = []
    for h in range(4):
        s0, s1 = h * BH, (h + 1) * BH
        gidx_h = jnp.concatenate([rows_c[s0:s1].reshape(-1),
                                  rows_w[s0:s1].reshape(-1)])
        kg, vg = _sc_gather(k2, v2, gidx_h)
        outs.append(_attn(seq_lens[s0:s1], q44[s0:s1], kn4[s0:s1], vn4[s0:s1],
                          knw, qnw, smat, gmat,
                          posf[s0 * HKV:s1 * HKV], wmat[s0:s1], kg, vg))
    o = jnp.concatenate(outs, axis=0)
    out = _outproj(o.reshape(B, DM), Wo, bo)
    return out[:, None, :]
